# Initial kernel scaffold; baseline (speedup 1.0000x reference)
import functools

import jax
import jax.numpy as jnp
from jax import lax
from jax.experimental import pallas as pl
from jax.experimental.pallas import tpu as pltpu
from jax.experimental.pallas import tpu_sc as plsc

N_NODES = 10000
FDIM = 128
NC = 2
NS = 16
NW = NC * NS
CH = 128
CNTW = 16

NACC = 10240
RPT = NACC // NS
DUMMY_ROW = N_NODES


def _spmm_body(with_count, table, gidx, sidx, *refs):
  if with_count:
    acc_out, cnt_out, rows_v, gi_v, si_v, ones_v, zc_v, acc_sh, cnt_sh, sem = refs
  else:
    acc_out, rows_v, gi_v, si_v, acc_sh, sem = refs

  c = lax.axis_index("c")
  s = lax.axis_index("s")
  zero16 = jnp.zeros((16,), jnp.float32)
  one16 = jnp.ones((16,), jnp.float32)

  def zero_row(i, _):
    for j in range(FDIM // 16):
      rows_v[i, pl.ds(j * 16, 16)] = zero16
    return 0
  lax.fori_loop(0, CH, zero_row, 0)

  if with_count:
    def fill_ones(i, _):
      ones_v[i, :] = one16
      return 0
    lax.fori_loop(0, CH, fill_ones, 0)

    def zero_cnt(i, _):
      zc_v[i, :] = zero16
      return 0
    lax.fori_loop(0, RPT, zero_cnt, 0)

  r0 = pl.multiple_of(s * RPT, CH)
  for k in range(RPT // CH):
    pltpu.sync_copy(rows_v, acc_sh.at[pl.ds(r0 + k * CH, CH)])
  if with_count:
    pltpu.sync_copy(zc_v, cnt_sh.at[pl.ds(r0, RPT)])
  plsc.subcore_barrier()

  ew = gidx.shape[0] // NW
  chunks = ew // CH
  base = (c * NS + s) * ew

  def step(i, _):
    off = pl.multiple_of(base + i * CH, CH)
    pltpu.sync_copy(gidx.at[pl.ds(off, CH)], gi_v)
    pltpu.sync_copy(sidx.at[pl.ds(off, CH)], si_v)
    pltpu.async_copy(table.at[gi_v], rows_v, sem).wait()
    pltpu.sync_copy(rows_v, acc_sh.at[si_v], add=True)
    if with_count:
      pltpu.sync_copy(ones_v, cnt_sh.at[si_v], add=True)
    return 0
  lax.fori_loop(0, chunks, step, 0)
  plsc.subcore_barrier()

  pltpu.sync_copy(acc_sh.at[pl.ds(r0, RPT)], acc_out.at[c, pl.ds(r0, RPT)])
  if with_count:
    pltpu.sync_copy(cnt_sh.at[pl.ds(r0, RPT)], cnt_out.at[c, pl.ds(r0, RPT)])


def _make_spmm(with_count):
  out_acc = jax.ShapeDtypeStruct((NC, NACC, FDIM), jnp.float32)
  out_cnt = jax.ShapeDtypeStruct((NC, NACC, CNTW), jnp.float32)
  scratch = [
      pltpu.VMEM((CH, FDIM), jnp.float32),
      pltpu.VMEM((CH,), jnp.int32),
      pltpu.VMEM((CH,), jnp.int32),
  ]
  if with_count:
    scratch += [
        pltpu.VMEM((CH, CNTW), jnp.float32),
        pltpu.VMEM((RPT, CNTW), jnp.float32),
    ]
  scratch += [pltpu.VMEM_SHARED((NACC, FDIM), jnp.float32)]
  if with_count:
    scratch += [pltpu.VMEM_SHARED((NACC, CNTW), jnp.float32)]
  scratch += [pltpu.SemaphoreType.DMA]
  return pl.kernel(
      functools.partial(_spmm_body, with_count),
      out_type=(out_acc, out_cnt) if with_count else out_acc,
      mesh=plsc.VectorSubcoreMesh(core_axis_name="c", subcore_axis_name="s"),
      scratch_types=scratch,
  )


_spmm_count = _make_spmm(True)
_spmm = _make_spmm(False)


def _dense_body(acc_ref, cnt_ref, x_ref, wl_ref, bl_ref, wr_ref, o_ref):
  acc = acc_ref[0] + acc_ref[1]
  cnt = cnt_ref[:, 0] + cnt_ref[:, CNTW]
  inv = 1.0 / jnp.maximum(cnt, 1.0)
  mean = acc * inv[:, None]
  y = jnp.dot(mean, wl_ref[...], preferred_element_type=jnp.float32)
  y = y + bl_ref[...]
  y = y + jnp.dot(x_ref[...], wr_ref[...], preferred_element_type=jnp.float32)
  o_ref[...] = jnp.maximum(y, 0.0)


_DR = 1000


def _dense(acc, cnt2, x, wl, bl, wr):
  n = x.shape[0]
  grid = n // _DR
  return pl.pallas_call(
      _dense_body,
      grid=(grid,),
      in_specs=[
          pl.BlockSpec((NC, _DR, FDIM), lambda i: (0, i, 0)),
          pl.BlockSpec((_DR, 2 * CNTW), lambda i: (i, 0)),
          pl.BlockSpec((_DR, FDIM), lambda i: (i, 0)),
          pl.BlockSpec((FDIM, FDIM), lambda i: (0, 0)),
          pl.BlockSpec((1, FDIM), lambda i: (0, 0)),
          pl.BlockSpec((FDIM, FDIM), lambda i: (0, 0)),
      ],
      out_specs=pl.BlockSpec((_DR, FDIM), lambda i: (i, 0)),
      out_shape=jax.ShapeDtypeStruct((n, FDIM), jnp.float32),
  )(acc, cnt2, x, wl, bl, wr)


def kernel(x_human, x_bacterial, edge_index,
           h1_Wl, h1_bl, h1_Wr, h2_Wl, h2_bl, h2_Wr,
           b1_Wl, b1_bl, b1_Wr, b2_Wl, b2_bl, b2_Wr):
  src = edge_index[0]
  dst = edge_index[1]
  e = src.shape[0]
  ep = -(-e // (NW * CH)) * (NW * CH)
  pad = ep - e
  gpad = jnp.zeros((pad,), jnp.int32)
  spad = jnp.full((pad,), DUMMY_ROW, jnp.int32)
  g_h = jnp.concatenate([dst, gpad])
  s_h = jnp.concatenate([src, spad])
  g_b = jnp.concatenate([src, gpad])
  s_b = jnp.concatenate([dst, spad])

  acc_h1, cnt_h = _spmm_count(x_human, g_h, s_h)
  acc_b1, cnt_b = _spmm_count(x_bacterial, g_b, s_b)

  def cnt2(cp):
    return jnp.moveaxis(cp[:, :N_NODES, :], 0, 1).reshape(N_NODES, 2 * CNTW)

  cnt_h2 = cnt2(cnt_h)
  cnt_b2 = cnt2(cnt_b)

  h1 = _dense(acc_h1, cnt_h2, x_human, h1_Wl, h1_bl.reshape(1, -1), h1_Wr)
  b1 = _dense(acc_b1, cnt_b2, x_bacterial, b1_Wl, b1_bl.reshape(1, -1), b1_Wr)

  acc_h2 = _spmm(h1, g_h, s_h)
  acc_b2 = _spmm(b1, g_b, s_b)

  h2 = _dense(acc_h2, cnt_h2, h1, h2_Wl, h2_bl.reshape(1, -1), h2_Wr)
  b2 = _dense(acc_b2, cnt_b2, b1, b2_Wl, b2_bl.reshape(1, -1), b2_Wr)
  return (h2, b2)


# speedup vs baseline: 4.7498x; 4.7498x over previous
import functools

import jax
import jax.numpy as jnp
from jax import lax
from jax.experimental import pallas as pl
from jax.experimental.pallas import tpu as pltpu
from jax.experimental.pallas import tpu_sc as plsc

N_NODES = 10000
FDIM = 128
HALF = FDIM // 2
NC = 2
NS = 16
CH = 128
CNTW = 16

NACC = 10240
RPT = NACC // NS
DUMMY_ROW = N_NODES


def _spmm_body(with_count, table, gidx, sidx, *refs):
  if with_count:
    acc_out, cnt_out, rows_v, gi_v, si_v, ones_v, zc_v, acc_sh, cnt_sh, sem = refs
  else:
    acc_out, rows_v, gi_v, si_v, acc_sh, sem = refs

  c = lax.axis_index("c")
  s = lax.axis_index("s")
  zero16 = jnp.zeros((16,), jnp.float32)
  one16 = jnp.ones((16,), jnp.float32)
  on_sc0 = c == 0

  def zero_row(i, _):
    for j in range(HALF // 16):
      rows_v[i, pl.ds(j * 16, 16)] = zero16
    return 0
  lax.fori_loop(0, CH, zero_row, 0)

  if with_count:
    @pl.when(on_sc0)
    def _():
      def fill_ones(i, _):
        ones_v[i, :] = one16
        return 0
      lax.fori_loop(0, CH, fill_ones, 0)

      def zero_cnt(i, _):
        zc_v[i, :] = zero16
        return 0
      lax.fori_loop(0, RPT, zero_cnt, 0)

  r0 = pl.multiple_of(s * RPT, CH)
  for k in range(RPT // CH):
    pltpu.sync_copy(rows_v, acc_sh.at[pl.ds(r0 + k * CH, CH)])
  if with_count:
    @pl.when(on_sc0)
    def _():
      pltpu.sync_copy(zc_v, cnt_sh.at[pl.ds(r0, RPT)])
  plsc.subcore_barrier()

  ew = gidx.shape[0] // NS
  chunks = ew // CH
  base = s * ew
  cvec = jnp.full((16,), 0, jnp.int32) + c

  def step(i, _):
    off = pl.multiple_of(base + i * CH, CH)
    pltpu.sync_copy(gidx.at[pl.ds(off, CH)], gi_v)
    pltpu.sync_copy(sidx.at[pl.ds(off, CH)], si_v)
    for j in range(CH // 16):
      t = gi_v[pl.ds(j * 16, 16)]
      gi_v[pl.ds(j * 16, 16)] = t + t + cvec
    pltpu.async_copy(table.at[gi_v], rows_v, sem).wait()
    pltpu.sync_copy(rows_v, acc_sh.at[si_v], add=True)
    if with_count:
      @pl.when(on_sc0)
      def _():
        pltpu.sync_copy(ones_v, cnt_sh.at[si_v], add=True)
    return 0
  lax.fori_loop(0, chunks, step, 0)
  plsc.subcore_barrier()

  pltpu.sync_copy(acc_sh.at[pl.ds(r0, RPT)], acc_out.at[c, pl.ds(r0, RPT)])
  if with_count:
    @pl.when(on_sc0)
    def _():
      pltpu.sync_copy(cnt_sh.at[pl.ds(r0, RPT)], cnt_out.at[pl.ds(r0, RPT)])


@functools.cache
def _make_spmm(with_count):
  out_acc = jax.ShapeDtypeStruct((NC, NACC, HALF), jnp.float32)
  out_cnt = jax.ShapeDtypeStruct((NACC, CNTW), jnp.float32)
  scratch = [
      pltpu.VMEM((CH, HALF), jnp.float32),
      pltpu.VMEM((CH,), jnp.int32),
      pltpu.VMEM((CH,), jnp.int32),
  ]
  if with_count:
    scratch += [
        pltpu.VMEM((CH, CNTW), jnp.float32),
        pltpu.VMEM((RPT, CNTW), jnp.float32),
    ]
  scratch += [pltpu.VMEM_SHARED((NACC, HALF), jnp.float32)]
  if with_count:
    scratch += [pltpu.VMEM_SHARED((NACC, CNTW), jnp.float32)]
  scratch += [pltpu.SemaphoreType.DMA]
  return pl.kernel(
      functools.partial(_spmm_body, with_count),
      out_type=(out_acc, out_cnt) if with_count else out_acc,
      mesh=plsc.VectorSubcoreMesh(core_axis_name="c", subcore_axis_name="s"),
      scratch_types=scratch,
      compiler_params=pltpu.CompilerParams(use_tc_tiling_on_sc=False),
  )


def _dense_body(acc_ref, cnt_ref, x_ref, wl_ref, bl_ref, wr_ref, o_ref):
  cnt = cnt_ref[:, 0]
  inv = 1.0 / jnp.maximum(cnt, 1.0)
  m0 = acc_ref[0] * inv[:, None]
  m1 = acc_ref[1] * inv[:, None]
  y = jnp.dot(m0, wl_ref[pl.ds(0, HALF), :], preferred_element_type=jnp.float32)
  y = y + jnp.dot(m1, wl_ref[pl.ds(HALF, HALF), :],
                  preferred_element_type=jnp.float32)
  y = y + bl_ref[...]
  y = y + jnp.dot(x_ref[...], wr_ref[...], preferred_element_type=jnp.float32)
  o_ref[...] = jnp.maximum(y, 0.0)


_DR = 1000


def _dense(acc, cnt, x, wl, bl, wr):
  n = x.shape[0]
  grid = n // _DR
  return pl.pallas_call(
      _dense_body,
      grid=(grid,),
      in_specs=[
          pl.BlockSpec((NC, _DR, HALF), lambda i: (0, i, 0)),
          pl.BlockSpec((_DR, CNTW), lambda i: (i, 0)),
          pl.BlockSpec((_DR, FDIM), lambda i: (i, 0)),
          pl.BlockSpec((FDIM, FDIM), lambda i: (0, 0)),
          pl.BlockSpec((1, FDIM), lambda i: (0, 0)),
          pl.BlockSpec((FDIM, FDIM), lambda i: (0, 0)),
      ],
      out_specs=pl.BlockSpec((_DR, FDIM), lambda i: (i, 0)),
      out_shape=jax.ShapeDtypeStruct((n, FDIM), jnp.float32),
  )(acc, cnt, x, wl, bl, wr)


def kernel(x_human, x_bacterial, edge_index,
           h1_Wl, h1_bl, h1_Wr, h2_Wl, h2_bl, h2_Wr,
           b1_Wl, b1_bl, b1_Wr, b2_Wl, b2_bl, b2_Wr):
  src = edge_index[0]
  dst = edge_index[1]
  e = src.shape[0]
  ep = -(-e // (NS * CH)) * (NS * CH)
  pad = ep - e
  gpad = jnp.zeros((pad,), jnp.int32)
  spad = jnp.full((pad,), DUMMY_ROW, jnp.int32)
  g_h = jnp.concatenate([dst, gpad])
  s_h = jnp.concatenate([src, spad])
  g_b = jnp.concatenate([src, gpad])
  s_b = jnp.concatenate([dst, spad])

  spmm_count = _make_spmm(True)
  spmm = _make_spmm(False)

  def half_view(x):
    return x.reshape(-1, HALF)

  acc_h1, cnt_h = spmm_count(half_view(x_human), g_h, s_h)
  acc_b1, cnt_b = spmm_count(half_view(x_bacterial), g_b, s_b)

  cnt_h = cnt_h[:N_NODES]
  cnt_b = cnt_b[:N_NODES]

  h1 = _dense(acc_h1, cnt_h, x_human, h1_Wl, h1_bl.reshape(1, -1), h1_Wr)
  b1 = _dense(acc_b1, cnt_b, x_bacterial, b1_Wl, b1_bl.reshape(1, -1), b1_Wr)

  acc_h2 = spmm(half_view(h1), g_h, s_h)
  acc_b2 = spmm(half_view(b1), g_b, s_b)

  h2 = _dense(acc_h2, cnt_h, h1, h2_Wl, h2_bl.reshape(1, -1), h2_Wr)
  b2 = _dense(acc_b2, cnt_b, b1, b2_Wl, b2_bl.reshape(1, -1), b2_Wr)
  return (h2, b2)
